# baseline (speedup 1.0000x reference)
import jax
import jax.numpy as jnp
from jax.experimental import pallas as pl
from jax.experimental.pallas import tpu as pltpu

_BM = 1024


def _router_body(x_ref, wt_ref, idx_ref, pw_ref):
    bm = x_ref.shape[0]
    e_dim = wt_ref.shape[1]
    logits = jnp.dot(x_ref[...], wt_ref[...],
                     preferred_element_type=jnp.float32)
    m = jnp.max(logits, axis=-1, keepdims=True)
    e = jnp.exp(logits - m)
    ones = jnp.ones((e_dim, e_dim), dtype=jnp.float32)
    s = jax.lax.dot_general(e, ones, (((1,), (0,)), ((), ())),
                            preferred_element_type=jnp.float32)
    pw_ref[...] = e * (1.0 / s)
    lane = jax.lax.broadcasted_iota(jnp.int32, (1, e_dim), 1)
    w2 = jax.lax.bitcast_convert_type((127 - lane) << 23, jnp.float32)
    v = jnp.where(logits == m, w2, 0.0)
    t = jax.lax.dot_general(v, ones, (((1,), (0,)), ((), ())),
                            preferred_element_type=jnp.float32)
    bits = jax.lax.bitcast_convert_type(t[:, :1], jnp.int32)
    idx = jnp.maximum(127 - (bits >> 23), 0)
    idx_ref[...] = idx


def kernel(x, W):
    M, K = x.shape
    E = W.shape[0]
    wt = W.T
    grid = (M // _BM,)
    idx, pw = pl.pallas_call(
        _router_body,
        grid=grid,
        in_specs=[
            pl.BlockSpec((_BM, K), lambda i: (i, 0)),
            pl.BlockSpec((K, E), lambda i: (0, 0)),
        ],
        out_specs=[
            pl.BlockSpec((_BM, 1), lambda i: (i, 0)),
            pl.BlockSpec((_BM, E), lambda i: (i, 0)),
        ],
        out_shape=[
            jax.ShapeDtypeStruct((M, 1), jnp.int32),
            jax.ShapeDtypeStruct((M, E), jnp.float32),
        ],
        compiler_params=pltpu.CompilerParams(
            dimension_semantics=("parallel",),
        ),
    )(x, wt)
    return idx.reshape((M,)), pw


# speedup vs baseline: 1.1167x; 1.1167x over previous
import jax
import jax.numpy as jnp
from jax.experimental import pallas as pl
from jax.experimental.pallas import tpu as pltpu

_BM = 1024


def _router_body(xa_ref, xb_ref, wa_ref, wb_ref, idx_ref, pw_ref):
    bm = xa_ref.shape[0]
    e_dim = wa_ref.shape[1]
    logits = jnp.dot(xa_ref[...], wa_ref[...],
                     preferred_element_type=jnp.float32)
    logits += jnp.dot(xb_ref[...], wb_ref[...],
                      preferred_element_type=jnp.float32)
    m = jnp.max(logits, axis=-1, keepdims=True)
    e = jnp.exp(logits - m)
    ones = jnp.ones((e_dim, e_dim), dtype=jnp.float32)
    s = jax.lax.dot_general(e, ones, (((1,), (0,)), ((), ())),
                            preferred_element_type=jnp.float32)
    pw_ref[...] = e * (1.0 / s)
    lane = jax.lax.broadcasted_iota(jnp.int32, (1, e_dim), 1)
    w2 = jax.lax.bitcast_convert_type((127 - lane) << 23, jnp.float32)
    v = jnp.where(logits == m, w2, 0.0)
    t = jax.lax.dot_general(v, ones, (((1,), (0,)), ((), ())),
                            preferred_element_type=jnp.float32)
    bits = jax.lax.bitcast_convert_type(t[:, :1], jnp.int32)
    idx = jnp.maximum(127 - (bits >> 23), 0)
    idx_ref[...] = idx.reshape((bm // 128, 128))


def kernel(x, W):
    M, K = x.shape
    E = W.shape[0]
    kh = K // 2
    wt = W.T
    grid = (M // _BM,)
    idx, pw = pl.pallas_call(
        _router_body,
        grid=grid,
        in_specs=[
            pl.BlockSpec((_BM, kh), lambda i: (i, 0)),
            pl.BlockSpec((_BM, kh), lambda i: (i, 1)),
            pl.BlockSpec((kh, E), lambda i: (0, 0)),
            pl.BlockSpec((kh, E), lambda i: (1, 0)),
        ],
        out_specs=[
            pl.BlockSpec((_BM // 128, 128), lambda i: (i, 0)),
            pl.BlockSpec((_BM, E), lambda i: (i, 0)),
        ],
        out_shape=[
            jax.ShapeDtypeStruct((M // 128, 128), jnp.int32),
            jax.ShapeDtypeStruct((M, E), jnp.float32),
        ],
        compiler_params=pltpu.CompilerParams(
            dimension_semantics=("parallel",),
        ),
    )(x, x, wt, wt)
    return idx.reshape((M,)), pw
